# Initial kernel scaffold; baseline (speedup 1.0000x reference)
import math

import jax
import jax.numpy as jnp
from jax.experimental import pallas as pl

DIM = 512
QS = 8
QT = 256
KS = 16
KT = 256
TOPK = 32
NEG_INF = float("-inf")


def _proj_kv_body(src_ref, wk_ref, bk_ref, wv_ref, bv_ref, k_ref, v_ref):
    x = src_ref[0]
    k_ref[0] = jnp.dot(x, wk_ref[...], preferred_element_type=jnp.float32) + bk_ref[...]
    v_ref[0] = jnp.dot(x, wv_ref[...], preferred_element_type=jnp.float32) + bv_ref[...]


def _attend_body(q_ref, ctx_ref, wc_ref, bc_ref, wq_ref, bq_ref,
                 k_ref, v_ref, mask_ref, wo_ref, bo_ref, out_ref):
    cond = jnp.dot(ctx_ref[...], wc_ref[...], preferred_element_type=jnp.float32) + bc_ref[...]
    gamma = cond[:, :DIM]
    beta = cond[:, DIM:2 * DIM]

    q = q_ref[0, 0]
    qm = q * (1.0 + gamma) + beta
    qp = jnp.dot(qm, wq_ref[...], preferred_element_type=jnp.float32) + bq_ref[...]

    kp = k_ref[0]
    scores = jax.lax.dot_general(
        qp, kp, (((1,), (1,)), ((), ())), preferred_element_type=jnp.float32)
    scores = scores * (1.0 / math.sqrt(DIM)) + mask_ref[...]

    row_max = jnp.max(scores, axis=1, keepdims=True)

    def body(_, carry):
        s, _ = carry
        m = jnp.max(s, axis=1, keepdims=True)
        s = jnp.where(s == m, NEG_INF, s)
        return s, m

    _, thresh = jax.lax.fori_loop(0, TOPK, body, (scores, row_max))

    w = jnp.where(scores >= thresh, jnp.exp(scores - row_max), 0.0)
    z = jnp.sum(w, axis=1, keepdims=True)
    r = jnp.dot(w, v_ref[0], preferred_element_type=jnp.float32) / z
    out_ref[0, 0] = jnp.dot(r, wo_ref[...], preferred_element_type=jnp.float32) + bo_ref[...]


def kernel(query, source, contexts_0, contexts_1, Wq, bq, Wk, bk, Wv, bv, Wo, bo, Wc, bc, mask):
    bsz = query.shape[0]
    src_flat = source.reshape(bsz, KS * KT, DIM)
    ctx = jnp.concatenate([contexts_0, contexts_1], axis=-1)

    kp, vp = pl.pallas_call(
        _proj_kv_body,
        grid=(bsz, KS),
        in_specs=[
            pl.BlockSpec((1, KT, DIM), lambda b, s: (b, s, 0)),
            pl.BlockSpec((DIM, DIM), lambda b, s: (0, 0)),
            pl.BlockSpec((DIM,), lambda b, s: (0,)),
            pl.BlockSpec((DIM, DIM), lambda b, s: (0, 0)),
            pl.BlockSpec((DIM,), lambda b, s: (0,)),
        ],
        out_specs=[
            pl.BlockSpec((1, KT, DIM), lambda b, s: (b, s, 0)),
            pl.BlockSpec((1, KT, DIM), lambda b, s: (b, s, 0)),
        ],
        out_shape=[
            jax.ShapeDtypeStruct((bsz, KS * KT, DIM), jnp.float32),
            jax.ShapeDtypeStruct((bsz, KS * KT, DIM), jnp.float32),
        ],
    )(src_flat, Wk, bk, Wv, bv)

    out = pl.pallas_call(
        _attend_body,
        grid=(bsz, QS),
        in_specs=[
            pl.BlockSpec((1, 1, QT, DIM), lambda b, t: (b, t, 0, 0)),
            pl.BlockSpec((1, 2 * DIM), lambda b, t: (b, 0)),
            pl.BlockSpec((2 * DIM, 3 * DIM), lambda b, t: (0, 0)),
            pl.BlockSpec((3 * DIM,), lambda b, t: (0,)),
            pl.BlockSpec((DIM, DIM), lambda b, t: (0, 0)),
            pl.BlockSpec((DIM,), lambda b, t: (0,)),
            pl.BlockSpec((1, KS * KT, DIM), lambda b, t: (b, 0, 0)),
            pl.BlockSpec((1, KS * KT, DIM), lambda b, t: (b, 0, 0)),
            pl.BlockSpec((QT, KS * KT), lambda b, t: (t, 0)),
            pl.BlockSpec((DIM, DIM), lambda b, t: (0, 0)),
            pl.BlockSpec((DIM,), lambda b, t: (0,)),
        ],
        out_specs=pl.BlockSpec((1, 1, QT, DIM), lambda b, t: (b, t, 0, 0)),
        out_shape=jax.ShapeDtypeStruct((bsz, QS, QT, DIM), jnp.float32),
    )(query, ctx, Wc, bc, Wq, bq, kp, vp, mask, Wo, bo)

    return out


# speedup vs baseline: 9.3221x; 9.3221x over previous
import math

import jax
import jax.numpy as jnp
from jax.experimental import pallas as pl

DIM = 512
QS = 8
QT = 256
KS = 16
KT = 256
TOPK = 32
NEG_INF = float("-inf")


def _proj_kv_body(src_ref, wk_ref, bk_ref, wv_ref, bv_ref, k_ref, v_ref):
    x = src_ref[0]
    k_ref[0] = jnp.dot(x, wk_ref[...], preferred_element_type=jnp.float32) + bk_ref[...]
    v_ref[0] = jnp.dot(x, wv_ref[...], preferred_element_type=jnp.float32) + bv_ref[...]


def _attend_body(q_ref, ctx_ref, wc_ref, bc_ref, wq_ref, bq_ref,
                 k_ref, v_ref, mask_ref, wo_ref, bo_ref, out_ref):
    cond = jnp.dot(ctx_ref[...], wc_ref[...], preferred_element_type=jnp.float32) + bc_ref[...]
    b = pl.program_id(0)
    rows = jax.lax.broadcasted_iota(jnp.int32, cond.shape, 0)
    cond_b = jnp.sum(jnp.where(rows == b, cond, 0.0), axis=0, keepdims=True)
    gamma = cond_b[:, :DIM]
    beta = cond_b[:, DIM:2 * DIM]

    q = q_ref[0, 0]
    qm = q * (1.0 + gamma) + beta
    qp = jnp.dot(qm, wq_ref[...], preferred_element_type=jnp.float32) + bq_ref[...]

    kp = k_ref[0]
    scores = jax.lax.dot_general(
        qp, kp, (((1,), (1,)), ((), ())), preferred_element_type=jnp.float32)
    scores = scores * (1.0 / math.sqrt(DIM)) + mask_ref[...]

    row_max = jnp.max(scores, axis=1, keepdims=True)

    def body(_, carry):
        s, _ = carry
        m = jnp.max(s, axis=1, keepdims=True)
        s = jnp.where(s == m, NEG_INF, s)
        return s, m

    _, thresh = jax.lax.fori_loop(0, TOPK, body, (scores, row_max))

    w = jnp.where(scores >= thresh, jnp.exp(scores - row_max), 0.0)
    z = jnp.sum(w, axis=1, keepdims=True)
    r = jnp.dot(w, v_ref[0], preferred_element_type=jnp.float32) / z
    out_ref[0, 0] = jnp.dot(r, wo_ref[...], preferred_element_type=jnp.float32) + bo_ref[...]


def kernel(query, source, contexts_0, contexts_1, Wq, bq, Wk, bk, Wv, bv, Wo, bo, Wc, bc, mask):
    bsz = query.shape[0]
    src_flat = source.reshape(bsz, KS * KT, DIM)
    ctx = jnp.concatenate([contexts_0, contexts_1], axis=-1)

    kp, vp = pl.pallas_call(
        _proj_kv_body,
        grid=(bsz, KS),
        in_specs=[
            pl.BlockSpec((1, KT, DIM), lambda b, s: (b, s, 0)),
            pl.BlockSpec((DIM, DIM), lambda b, s: (0, 0)),
            pl.BlockSpec((DIM,), lambda b, s: (0,)),
            pl.BlockSpec((DIM, DIM), lambda b, s: (0, 0)),
            pl.BlockSpec((DIM,), lambda b, s: (0,)),
        ],
        out_specs=[
            pl.BlockSpec((1, KT, DIM), lambda b, s: (b, s, 0)),
            pl.BlockSpec((1, KT, DIM), lambda b, s: (b, s, 0)),
        ],
        out_shape=[
            jax.ShapeDtypeStruct((bsz, KS * KT, DIM), jnp.float32),
            jax.ShapeDtypeStruct((bsz, KS * KT, DIM), jnp.float32),
        ],
    )(src_flat, Wk, bk, Wv, bv)

    out = pl.pallas_call(
        _attend_body,
        grid=(bsz, QS),
        in_specs=[
            pl.BlockSpec((1, 1, QT, DIM), lambda b, t: (b, t, 0, 0)),
            pl.BlockSpec((bsz, 2 * DIM), lambda b, t: (0, 0)),
            pl.BlockSpec((2 * DIM, 3 * DIM), lambda b, t: (0, 0)),
            pl.BlockSpec((3 * DIM,), lambda b, t: (0,)),
            pl.BlockSpec((DIM, DIM), lambda b, t: (0, 0)),
            pl.BlockSpec((DIM,), lambda b, t: (0,)),
            pl.BlockSpec((1, KS * KT, DIM), lambda b, t: (b, 0, 0)),
            pl.BlockSpec((1, KS * KT, DIM), lambda b, t: (b, 0, 0)),
            pl.BlockSpec((QT, KS * KT), lambda b, t: (t, 0)),
            pl.BlockSpec((DIM, DIM), lambda b, t: (0, 0)),
            pl.BlockSpec((DIM,), lambda b, t: (0,)),
        ],
        out_specs=pl.BlockSpec((1, 1, QT, DIM), lambda b, t: (b, t, 0, 0)),
        out_shape=jax.ShapeDtypeStruct((bsz, QS, QT, DIM), jnp.float32),
    )(query, ctx, Wc, bc, Wq, bq, kp, vp, mask, Wo, bo)

    return out


# speedup vs baseline: 26.5870x; 2.8520x over previous
import math

import jax
import jax.numpy as jnp
import numpy as np
from jax.experimental import pallas as pl

DIM = 512
QS = 8
QT = 256
KS = 16
KT = 256
TOPK = 32
WSTEPS = 5
WCOLS = WSTEPS * KT
NEG_INF = float("-inf")


def _proj_kv_body(src_ref, wk_ref, bk_ref, wv_ref, bv_ref, k_ref, v_ref):
    x = src_ref[0]
    k_ref[0] = jnp.dot(x, wk_ref[...], preferred_element_type=jnp.float32) + bk_ref[...]
    v_ref[0] = jnp.dot(x, wv_ref[...], preferred_element_type=jnp.float32) + bv_ref[...]


def _band_start(t):
    c = (2 * (KS - 1) * t + (QS - 1)) // (2 * (QS - 1))
    s = jnp.clip(c - 2, 0, KS - WSTEPS)
    return s * KT


def _attend_body(q_ref, ctx_ref, wc_ref, bc_ref, wq_ref, bq_ref,
                 k_ref, v_ref, mask_ref, wo_ref, bo_ref, out_ref):
    cond = jnp.dot(ctx_ref[...], wc_ref[...], preferred_element_type=jnp.float32) + bc_ref[...]
    b = pl.program_id(0)
    rows = jax.lax.broadcasted_iota(jnp.int32, cond.shape, 0)
    cond_b = jnp.sum(jnp.where(rows == b, cond, 0.0), axis=0, keepdims=True)
    gamma = cond_b[:, :DIM]
    beta = cond_b[:, DIM:2 * DIM]

    q = q_ref[0, 0]
    qm = q * (1.0 + gamma) + beta
    qp = (jnp.dot(qm, wq_ref[...], preferred_element_type=jnp.float32)
          + bq_ref[...]) * (1.0 / math.sqrt(DIM))

    start = _band_start(pl.program_id(1))
    kb = k_ref[0, pl.ds(start, WCOLS), :]
    scores = jax.lax.dot_general(
        qp, kb, (((1,), (1,)), ((), ())), preferred_element_type=jnp.float32)
    scores = scores + mask_ref[0]

    row_max = jnp.max(scores, axis=1, keepdims=True)

    def body(_, carry):
        s, _ = carry
        m = jnp.max(s, axis=1, keepdims=True)
        s = jnp.where(s == m, NEG_INF, s)
        return s, m

    _, thresh = jax.lax.fori_loop(0, TOPK, body, (scores, row_max))

    w = jnp.where(scores >= thresh, jnp.exp(scores - row_max), 0.0)
    z = jnp.sum(w, axis=1, keepdims=True)
    vb = v_ref[0, pl.ds(start, WCOLS), :]
    r = jnp.dot(w, vb, preferred_element_type=jnp.float32) / z
    out_ref[0, 0] = jnp.dot(r, wo_ref[...], preferred_element_type=jnp.float32) + bo_ref[...]


def kernel(query, source, contexts_0, contexts_1, Wq, bq, Wk, bk, Wv, bv, Wo, bo, Wc, bc, mask):
    bsz = query.shape[0]
    src_flat = source.reshape(bsz, KS * KT, DIM)
    ctx = jnp.concatenate([contexts_0, contexts_1], axis=-1)

    centers = np.round(np.linspace(0, KS - 1, QS)).astype(np.int64)
    starts = np.clip(centers - 2, 0, KS - WSTEPS) * KT
    mask_band = jnp.stack(
        [jax.lax.dynamic_slice(mask[t * QT], (int(starts[t]),), (WCOLS,))
         for t in range(QS)]).reshape(QS, 1, WCOLS)

    kp, vp = pl.pallas_call(
        _proj_kv_body,
        grid=(bsz, KS),
        in_specs=[
            pl.BlockSpec((1, KT, DIM), lambda b, s: (b, s, 0)),
            pl.BlockSpec((DIM, DIM), lambda b, s: (0, 0)),
            pl.BlockSpec((DIM,), lambda b, s: (0,)),
            pl.BlockSpec((DIM, DIM), lambda b, s: (0, 0)),
            pl.BlockSpec((DIM,), lambda b, s: (0,)),
        ],
        out_specs=[
            pl.BlockSpec((1, KT, DIM), lambda b, s: (b, s, 0)),
            pl.BlockSpec((1, KT, DIM), lambda b, s: (b, s, 0)),
        ],
        out_shape=[
            jax.ShapeDtypeStruct((bsz, KS * KT, DIM), jnp.float32),
            jax.ShapeDtypeStruct((bsz, KS * KT, DIM), jnp.float32),
        ],
    )(src_flat, Wk, bk, Wv, bv)

    out = pl.pallas_call(
        _attend_body,
        grid=(bsz, QS),
        in_specs=[
            pl.BlockSpec((1, 1, QT, DIM), lambda b, t: (b, t, 0, 0)),
            pl.BlockSpec((bsz, 2 * DIM), lambda b, t: (0, 0)),
            pl.BlockSpec((2 * DIM, 3 * DIM), lambda b, t: (0, 0)),
            pl.BlockSpec((3 * DIM,), lambda b, t: (0,)),
            pl.BlockSpec((DIM, DIM), lambda b, t: (0, 0)),
            pl.BlockSpec((DIM,), lambda b, t: (0,)),
            pl.BlockSpec((1, KS * KT, DIM), lambda b, t: (b, 0, 0)),
            pl.BlockSpec((1, KS * KT, DIM), lambda b, t: (b, 0, 0)),
            pl.BlockSpec((1, 1, WCOLS), lambda b, t: (t, 0, 0)),
            pl.BlockSpec((DIM, DIM), lambda b, t: (0, 0)),
            pl.BlockSpec((DIM,), lambda b, t: (0,)),
        ],
        out_specs=pl.BlockSpec((1, 1, QT, DIM), lambda b, t: (b, t, 0, 0)),
        out_shape=jax.ShapeDtypeStruct((bsz, QS, QT, DIM), jnp.float32),
    )(query, ctx, Wc, bc, Wq, bq, kp, vp, mask_band, Wo, bo)

    return out
